# Initial kernel scaffold; baseline (speedup 1.0000x reference)
import jax
import jax.numpy as jnp
from jax.experimental import pallas as pl


def kernel(h, idx):
    raise NotImplementedError("write your pallas kernel here")


# speedup vs baseline: 1.4781x; 1.4781x over previous
import functools

import jax
import jax.numpy as jnp
from jax import lax
from jax.experimental import pallas as pl
from jax.experimental.pallas import tpu as pltpu
from jax.experimental.pallas import tpu_sc as plsc

N_NODES = 10000
D = 128
E = 320000
R = 2.0
T = 1.0
EPS = 1e-6

NC = 2
NS = 16
NW = NC * NS
E_PER_W = E // NW
CB = 400
NCHUNK = E_PER_W // CB
GSUB = 80
NSUB = CB // GSUB
NG = CB // 16
DUNROLL = 32


def _sc_body(h_hbm, idx0_hbm, idx1_hbm, out_hbm,
             idx0_v, idx1_v, rows0_v, rows1_v, out_v, sem0, sem1):
    c = lax.axis_index("c")
    s = lax.axis_index("s")
    wid = s * NC + c
    lane = lax.iota(jnp.int32, 16)

    def chunk_body(ci, carry):
        base = wid * E_PER_W + ci * CB
        pltpu.sync_copy(idx0_hbm.at[pl.ds(base, CB)], idx0_v)
        pltpu.sync_copy(idx1_hbm.at[pl.ds(base, CB)], idx1_v)
        cps = []
        for j in range(NSUB):
            sl = pl.ds(j * GSUB, GSUB)
            cps.append(pltpu.async_copy(
                h_hbm.at[idx0_v.at[sl]], rows0_v.at[sl], sem0))
            cps.append(pltpu.async_copy(
                h_hbm.at[idx1_v.at[sl]], rows1_v.at[sl], sem1))
        for cp in cps:
            cp.wait()

        def group_body(g, gcarry):
            e_idx = g * 16 + lane
            zero16 = jnp.zeros((16,), jnp.int32)
            a0 = plsc.load_gather(rows0_v, [e_idx, zero16])
            b0 = plsc.load_gather(rows1_v, [e_idx, zero16])
            p00 = a0 * b0

            zf = jnp.zeros((16,), jnp.float32)

            def dot_body(k, carry):
                d_idx, acc0, acc1, acc2, acc3 = carry
                accs = [acc0, acc1, acc2, acc3]
                di = d_idx
                for u in range(DUNROLL):
                    a = plsc.load_gather(rows0_v, [e_idx, di])
                    b = plsc.load_gather(rows1_v, [e_idx, di])
                    accs[u % 4] = accs[u % 4] + a * b
                    di = di + 1
                return (di, accs[0], accs[1], accs[2], accs[3])

            _, s0, s1, s2, s3 = lax.fori_loop(
                0, D // DUNROLL, dot_body, (zero16, zf, zf, zf, zf))
            total = (s0 + s1) + (s2 + s3)
            out_v[pl.ds(g * 16, 16)] = p00 + p00 - total
            return gcarry

        lax.fori_loop(0, NG, group_body, 0)
        pltpu.sync_copy(out_v, out_hbm.at[pl.ds(base, CB)])
        return carry

    lax.fori_loop(0, NCHUNK, chunk_body, 0)


_sc_neg_ldot = functools.partial(
    pl.kernel,
    mesh=plsc.VectorSubcoreMesh(core_axis_name="c", subcore_axis_name="s"),
    out_type=jax.ShapeDtypeStruct((E,), jnp.float32),
    compiler_params=pltpu.CompilerParams(needs_layout_passes=False),
    scratch_types=[
        pltpu.VMEM((CB,), jnp.int32),
        pltpu.VMEM((CB,), jnp.int32),
        pltpu.VMEM((CB, D), jnp.float32),
        pltpu.VMEM((CB, D), jnp.float32),
        pltpu.VMEM((CB,), jnp.float32),
        pltpu.SemaphoreType.DMA,
        pltpu.SemaphoreType.DMA,
    ],
)(_sc_body)


def _tail_body(x_ref, o_ref):
    x = jnp.maximum(x_ref[...], 1.0 + EPS)
    d = jnp.log(x + jnp.sqrt((x - 1.0) * (x + 1.0)))
    sq = d * d
    o_ref[...] = 1.0 / (jnp.exp((sq - R) / T) + 1.0)


_tail = pl.pallas_call(
    _tail_body,
    out_shape=jax.ShapeDtypeStruct((E // D, D), jnp.float32),
)


@jax.jit
def kernel(h, idx):
    idx0 = idx[:, 0]
    idx1 = idx[:, 1]
    neg_ldot = _sc_neg_ldot(h, idx0, idx1)
    probs = _tail(neg_ldot.reshape(E // D, D))
    return probs.reshape(E)


# speedup vs baseline: 3.9249x; 2.6554x over previous
import functools

import jax
import jax.numpy as jnp
from jax import lax
from jax.experimental import pallas as pl
from jax.experimental.pallas import tpu as pltpu
from jax.experimental.pallas import tpu_sc as plsc

N_NODES = 10000
D = 128
E = 320000
R = 2.0
T = 1.0
EPS = 1e-6

NC = 2
NS = 16
NW = NC * NS
E_PER_W = E // NW
CB = 400
NCHUNK = E_PER_W // CB
GSUB = 80
NSUB = CB // GSUB
NG = CB // 16
DUNROLL = 32


def _sc_body(h_hbm, idx0_hbm, idx1_hbm, out_hbm,
             idx0_v, idx1_v, rows0_v, rows1_v, out_v, sem0, sem1):
    c = lax.axis_index("c")
    s = lax.axis_index("s")
    wid = s * NC + c
    lane = lax.iota(jnp.int32, 16)

    def chunk_body(ci, carry):
        base = wid * E_PER_W + ci * CB
        pltpu.sync_copy(idx0_hbm.at[pl.ds(base, CB)], idx0_v)
        pltpu.sync_copy(idx1_hbm.at[pl.ds(base, CB)], idx1_v)
        cps = []
        for j in range(NSUB):
            sl = pl.ds(j * GSUB, GSUB)
            cps.append(pltpu.async_copy(
                h_hbm.at[idx0_v.at[sl]], rows0_v.at[sl], sem0))
            cps.append(pltpu.async_copy(
                h_hbm.at[idx1_v.at[sl]], rows1_v.at[sl], sem1))
        for cp in cps:
            cp.wait()

        two_e0 = jnp.where(lane == 0, 2.0, 0.0).astype(jnp.float32)

        def group_body(g, gcarry):
            base_e = g * 16
            res = jnp.zeros((16,), jnp.float32)
            for e in range(16):
                ei = base_e + e
                p = [rows0_v[ei, pl.ds(16 * j, 16)]
                     * rows1_v[ei, pl.ds(16 * j, 16)] for j in range(8)]
                s = ((p[0] + p[1]) + (p[2] + p[3])) \
                    + ((p[4] + p[5]) + (p[6] + p[7]))
                s = s - p[0] * two_e0
                res = jnp.where(lane == e, -jnp.sum(s), res)
            out_v[pl.ds(base_e, 16)] = res
            return gcarry

        lax.fori_loop(0, NG, group_body, 0)
        pltpu.sync_copy(out_v, out_hbm.at[pl.ds(base, CB)])
        return carry

    lax.fori_loop(0, NCHUNK, chunk_body, 0)


_sc_neg_ldot = functools.partial(
    pl.kernel,
    mesh=plsc.VectorSubcoreMesh(core_axis_name="c", subcore_axis_name="s"),
    out_type=jax.ShapeDtypeStruct((E,), jnp.float32),
    compiler_params=pltpu.CompilerParams(needs_layout_passes=False),
    scratch_types=[
        pltpu.VMEM((CB,), jnp.int32),
        pltpu.VMEM((CB,), jnp.int32),
        pltpu.VMEM((CB, D), jnp.float32),
        pltpu.VMEM((CB, D), jnp.float32),
        pltpu.VMEM((CB,), jnp.float32),
        pltpu.SemaphoreType.DMA,
        pltpu.SemaphoreType.DMA,
    ],
)(_sc_body)


def _tail_body(x_ref, o_ref):
    x = jnp.maximum(x_ref[...], 1.0 + EPS)
    d = jnp.log(x + jnp.sqrt((x - 1.0) * (x + 1.0)))
    sq = d * d
    o_ref[...] = 1.0 / (jnp.exp((sq - R) / T) + 1.0)


_tail = pl.pallas_call(
    _tail_body,
    out_shape=jax.ShapeDtypeStruct((E // D, D), jnp.float32),
)


@jax.jit
def kernel(h, idx):
    idx0 = idx[:, 0]
    idx1 = idx[:, 1]
    neg_ldot = _sc_neg_ldot(h, idx0, idx1)
    probs = _tail(neg_ldot.reshape(E // D, D))
    return probs.reshape(E)


# speedup vs baseline: 5.2133x; 1.3282x over previous
import functools

import jax
import jax.numpy as jnp
from jax import lax
from jax.experimental import pallas as pl
from jax.experimental.pallas import tpu as pltpu
from jax.experimental.pallas import tpu_sc as plsc

N_NODES = 10000
D = 128
E = 320000
R = 2.0
T = 1.0
EPS = 1e-6

NC = 2
NS = 16
NW = NC * NS
E_PER_W = E // NW
CB = 80
NCHUNK = E_PER_W // CB
NG = CB // 16
NPAIR = (NCHUNK + 1) // 2


def _sc_body(h_hbm, idx0_hbm, idx1_hbm, out_hbm,
             idx0_w, idx1_w, rows0_v, rows1_v, out_w,
             s00, s10, s01, s11):
    c = lax.axis_index("c")
    s = lax.axis_index("s")
    wid = s * NC + c
    lane = lax.iota(jnp.int32, 16)
    two_e0 = jnp.where(lane == 0, 2.0, 0.0).astype(jnp.float32)
    sems = ((s00, s10), (s01, s11))

    pltpu.sync_copy(idx0_hbm.at[wid], idx0_w)
    pltpu.sync_copy(idx1_hbm.at[wid], idx1_w)

    def fire(ci, b):
        pltpu.make_async_copy(
            h_hbm.at[idx0_w.at[ci]], rows0_v.at[b], sems[b][0]).start()
        pltpu.make_async_copy(
            h_hbm.at[idx1_w.at[ci]], rows1_v.at[b], sems[b][1]).start()

    def wait(b):
        pltpu.make_async_copy(
            h_hbm.at[idx0_w.at[0]], rows0_v.at[b], sems[b][0]).wait()
        pltpu.make_async_copy(
            h_hbm.at[idx1_w.at[0]], rows1_v.at[b], sems[b][1]).wait()

    def compute(ci, b):
        r0 = rows0_v.at[b]
        r1 = rows1_v.at[b]

        def group_body(g, gcarry):
            base_e = g * 16
            res = jnp.zeros((16,), jnp.float32)
            for e in range(16):
                ei = base_e + e
                p = [r0[ei, pl.ds(16 * j, 16)]
                     * r1[ei, pl.ds(16 * j, 16)] for j in range(8)]
                t = ((p[0] + p[1]) + (p[2] + p[3])) \
                    + ((p[4] + p[5]) + (p[6] + p[7]))
                t = t - p[0] * two_e0
                res = jnp.where(lane == e, -jnp.sum(t), res)
            out_w[ci, pl.ds(base_e, 16)] = res
            return gcarry

        lax.fori_loop(0, NG, group_body, 0)

    fire(0, 0)

    def pair_body(k, carry):
        for b in range(2):
            ci = 2 * k + b

            @pl.when(ci < NCHUNK)
            def _():
                wait(b)

                @pl.when(ci + 1 < NCHUNK)
                def _():
                    fire(ci + 1, 1 - b)

                compute(ci, b)
        return carry

    lax.fori_loop(0, NPAIR, pair_body, 0)
    pltpu.sync_copy(out_w, out_hbm.at[wid])


_sc_neg_ldot = functools.partial(
    pl.kernel,
    mesh=plsc.VectorSubcoreMesh(core_axis_name="c", subcore_axis_name="s"),
    out_type=jax.ShapeDtypeStruct((NW, NCHUNK, CB), jnp.float32),
    compiler_params=pltpu.CompilerParams(needs_layout_passes=False),
    scratch_types=[
        pltpu.VMEM((NCHUNK, CB), jnp.int32),
        pltpu.VMEM((NCHUNK, CB), jnp.int32),
        pltpu.VMEM((2, CB, D), jnp.float32),
        pltpu.VMEM((2, CB, D), jnp.float32),
        pltpu.VMEM((NCHUNK, CB), jnp.float32),
        pltpu.SemaphoreType.DMA,
        pltpu.SemaphoreType.DMA,
        pltpu.SemaphoreType.DMA,
        pltpu.SemaphoreType.DMA,
    ],
)(_sc_body)


def _tail_body(x_ref, o_ref):
    x = jnp.maximum(x_ref[...], 1.0 + EPS)
    d = jnp.log(x + jnp.sqrt((x - 1.0) * (x + 1.0)))
    sq = d * d
    o_ref[...] = 1.0 / (jnp.exp((sq - R) / T) + 1.0)


_tail = pl.pallas_call(
    _tail_body,
    out_shape=jax.ShapeDtypeStruct((E // D, D), jnp.float32),
)


@jax.jit
def kernel(h, idx):
    idx0 = idx[:, 0].reshape(NW, NCHUNK, CB)
    idx1 = idx[:, 1].reshape(NW, NCHUNK, CB)
    neg_ldot = _sc_neg_ldot(h, idx0, idx1)
    probs = _tail(neg_ldot.reshape(E // D, D))
    return probs.reshape(E)


# speedup vs baseline: 9.4597x; 1.8145x over previous
import functools

import jax
import jax.numpy as jnp
from jax import lax
from jax.experimental import pallas as pl
from jax.experimental.pallas import tpu as pltpu
from jax.experimental.pallas import tpu_sc as plsc

N_NODES = 10000
D = 128
E = 320000
R = 2.0
T = 1.0
EPS = 1e-6

NC = 2
NS = 16
NW = NC * NS
E_PER_W = E // NW
CB = 80
NCHUNK = E_PER_W // CB
NG = CB // 16
NPAIR = (NCHUNK + 1) // 2


def _sc_body(h_hbm, idx0_hbm, idx1_hbm, out_hbm,
             idx0_w, idx1_w, rows0_v, rows1_v, out_w,
             s00, s10, s01, s11):
    c = lax.axis_index("c")
    s = lax.axis_index("s")
    wid = s * NC + c
    lane = lax.iota(jnp.int32, 16)
    two_e0 = jnp.where(lane == 0, 2.0, 0.0).astype(jnp.float32)
    sems = ((s00, s10), (s01, s11))

    pltpu.sync_copy(idx0_hbm.at[wid], idx0_w)
    pltpu.sync_copy(idx1_hbm.at[wid], idx1_w)

    def fire(ci, b):
        pltpu.make_async_copy(
            h_hbm.at[idx0_w.at[ci]], rows0_v.at[b], sems[b][0]).start()
        pltpu.make_async_copy(
            h_hbm.at[idx1_w.at[ci]], rows1_v.at[b], sems[b][1]).start()

    def wait(b):
        pltpu.make_async_copy(
            h_hbm.at[idx0_w.at[0]], rows0_v.at[b], sems[b][0]).wait()
        pltpu.make_async_copy(
            h_hbm.at[idx1_w.at[0]], rows1_v.at[b], sems[b][1]).wait()

    lane15 = lane == 15

    def compute(ci, b):
        r0 = rows0_v.at[b]
        r1 = rows1_v.at[b]
        ci_v = jnp.full((16,), 0, jnp.int32) + ci

        def group_body(g, gcarry):
            base_e = g * 16
            base_v = jnp.full((16,), 0, jnp.int32) + base_e

            def load(e):
                ei = base_e + e
                return ([r0[ei, pl.ds(16 * j, 16)] for j in range(8)],
                        [r1[ei, pl.ds(16 * j, 16)] for j in range(8)])

            def emit(e, ab):
                a, b = ab
                p = [a[j] * b[j] for j in range(8)]
                t = ((p[0] + p[1]) + (p[2] + p[3])) \
                    + ((p[4] + p[5]) + (p[6] + p[7]))
                t = t - p[0] * two_e0
                cs = plsc.cumsum(-t)
                plsc.store_scatter(out_w, [ci_v, base_v + e], cs,
                                   mask=lane15)

            b0 = load(0)
            b1 = load(1)
            for e in range(2, 16):
                b2 = load(e)
                emit(e - 2, b0)
                b0, b1 = b1, b2
            emit(14, b0)
            emit(15, b1)
            return gcarry

        lax.fori_loop(0, NG, group_body, 0)

    fire(0, 0)

    def pair_body(k, carry):
        for b in range(2):
            ci = 2 * k + b

            @pl.when(ci < NCHUNK)
            def _():
                wait(b)

                @pl.when(ci + 1 < NCHUNK)
                def _():
                    fire(ci + 1, 1 - b)

                compute(ci, b)
        return carry

    lax.fori_loop(0, NPAIR, pair_body, 0)
    pltpu.sync_copy(out_w, out_hbm.at[wid])


_sc_neg_ldot = functools.partial(
    pl.kernel,
    mesh=plsc.VectorSubcoreMesh(core_axis_name="c", subcore_axis_name="s"),
    out_type=jax.ShapeDtypeStruct((NW, NCHUNK, CB), jnp.float32),
    compiler_params=pltpu.CompilerParams(needs_layout_passes=False),
    scratch_types=[
        pltpu.VMEM((NCHUNK, CB), jnp.int32),
        pltpu.VMEM((NCHUNK, CB), jnp.int32),
        pltpu.VMEM((2, CB, D), jnp.float32),
        pltpu.VMEM((2, CB, D), jnp.float32),
        pltpu.VMEM((NCHUNK, CB), jnp.float32),
        pltpu.SemaphoreType.DMA,
        pltpu.SemaphoreType.DMA,
        pltpu.SemaphoreType.DMA,
        pltpu.SemaphoreType.DMA,
    ],
)(_sc_body)


def _tail_body(x_ref, o_ref):
    x = jnp.maximum(x_ref[...], 1.0 + EPS)
    d = jnp.log(x + jnp.sqrt((x - 1.0) * (x + 1.0)))
    sq = d * d
    o_ref[...] = 1.0 / (jnp.exp((sq - R) / T) + 1.0)


_tail = pl.pallas_call(
    _tail_body,
    out_shape=jax.ShapeDtypeStruct((E // D, D), jnp.float32),
)


@jax.jit
def kernel(h, idx):
    idx0 = idx[:, 0].reshape(NW, NCHUNK, CB)
    idx1 = idx[:, 1].reshape(NW, NCHUNK, CB)
    neg_ldot = _sc_neg_ldot(h, idx0, idx1)
    probs = _tail(neg_ldot.reshape(E // D, D))
    return probs.reshape(E)
